# baseline (speedup 1.0000x reference)
import functools

import jax
import jax.numpy as jnp
from jax import lax
from jax.experimental import pallas as pl
from jax.experimental.pallas import tpu as pltpu
from jax.experimental.pallas import tpu_sc as plsc

VOCAB = 256
DIM = 64

NC = 2
NS = 16
NW = NC * NS
REP = 16

CHUNK = 32
NBUF = 8


def _table_body(embed_t_ref, w_t_ref, b_ref, t_ref):
    t = (
        lax.dot_general(
            embed_t_ref[...],
            w_t_ref[...],
            (((0,), (0,)), ((), ())),
            preferred_element_type=jnp.float32,
        )
        + b_ref[...]
    )
    t_ref[...] = jnp.broadcast_to(t[None], t_ref.shape)


def _make_table(embed, W, b):
    return pl.pallas_call(
        _table_body,
        out_shape=jax.ShapeDtypeStruct((REP, VOCAB, VOCAB), jnp.float32),
    )(embed.T, W.T, b.reshape(1, VOCAB))


def _make_gather(n_tokens):
    assert n_tokens % (NW * CHUNK) == 0
    bpw = n_tokens // NW
    nchunk = bpw // CHUNK
    assert nchunk % NBUF == 0

    mesh = plsc.VectorSubcoreMesh(core_axis_name="c", subcore_axis_name="s")

    qpg = 4
    grows = qpg * CHUNK
    ngrp = bpw // grows
    assert ngrp % 2 == 0

    @functools.partial(
        pl.kernel,
        mesh=mesh,
        out_type=jax.ShapeDtypeStruct((n_tokens, VOCAB), jnp.float32),
        scratch_types=[
            pltpu.VMEM((bpw,), jnp.int32),
            pltpu.VMEM((grows, VOCAB), jnp.float32),
            pltpu.VMEM((grows, VOCAB), jnp.float32),
            pltpu.SemaphoreType.DMA,
            pltpu.SemaphoreType.DMA,
            pltpu.SemaphoreType.DMA,
            pltpu.SemaphoreType.DMA,
        ],
    )
    def gather(table_hbm, idx_hbm, out_hbm, idx_v, b0, b1, gs0, gs1, ws0, ws1):
        wid = lax.axis_index("s") * NC + lax.axis_index("c")
        base = wid * bpw
        tpr = idx_hbm.shape[1] // bpw
        pltpu.sync_copy(
            idx_hbm.at[wid // tpr, pl.ds(lax.rem(wid, tpr) * bpw, bpw)], idx_v
        )
        tbl = table_hbm.at[lax.rem(wid, REP)]

        def gcopy(grp, q, buf, sem):
            return pltpu.make_async_copy(
                tbl.at[idx_v.at[pl.ds((grp * qpg + q) * CHUNK, CHUNK)]],
                buf.at[pl.ds(q * CHUNK, CHUNK)],
                sem,
            )

        def fire_gathers(grp, buf, sem):
            for q in range(qpg):
                gcopy(grp, q, buf, sem).start()

        def wait_gathers(grp, buf, sem):
            for q in range(qpg):
                gcopy(grp, q, buf, sem).wait()

        def wcopy(grp, buf, sem):
            return pltpu.make_async_copy(
                buf, out_hbm.at[pl.ds(base + grp * grows, grows)], sem
            )

        fire_gathers(0, b0, gs0)

        def outer(h, _):
            ga = 2 * h
            gb = 2 * h + 1

            @pl.when(h >= 1)
            def _():
                wcopy(gb - 2, b1, ws1).wait()

            fire_gathers(gb, b1, gs1)
            wait_gathers(ga, b0, gs0)
            wcopy(ga, b0, ws0).start()

            @pl.when(ga + 2 < ngrp)
            def _():
                wcopy(ga, b0, ws0).wait()
                fire_gathers(ga + 2, b0, gs0)

            wait_gathers(gb, b1, gs1)
            wcopy(gb, b1, ws1).start()
            return 0

        lax.fori_loop(0, ngrp // 2, outer, 0, unroll=False)
        wcopy(ngrp - 2, b0, ws0).wait()
        wcopy(ngrp - 1, b1, ws1).wait()

    return gather


def kernel(x, embed, W, b):
    batch, seq = x.shape
    n_tokens = batch * seq
    table = _make_table(embed, W, b)
    out = _make_gather(n_tokens)(table, x)
    return out.reshape(batch, seq, VOCAB)
